# baseline (speedup 1.0000x reference)
import functools

import jax
import jax.numpy as jnp
from jax import lax
from jax.experimental import pallas as pl
from jax.experimental.pallas import tpu as pltpu
from jax.experimental.pallas import tpu_sc as plsc

NSEG = 27
NPAIR = 20000
UNITS = 128
ROWS = 512
COLS = 1024

NCORE = 2
NSUB = 16
SEG_PER_CORE = 14
NCHUNK = 10
CHUNK = 125
PROW = 2 * CHUNK
IROW = 130
NLANE = 16
NVEC = UNITS // NLANE
NBUF = 5
ROW_UNROLL = 5
IVEC = 8


def _take16(vec, idx16):
    dnums = lax.GatherDimensionNumbers(
        offset_dims=(), collapsed_slice_dims=(0,), start_index_map=(0,))
    return lax.gather(vec, idx16[:, None], dnums, (1,),
                      mode=lax.GatherScatterMode.PROMISE_IN_BOUNDS)


def _sc_body(table_hbm, pairs_hbm, out_hbm,
             prs_v, idx_v, b0, b1, b2, b3, b4, acc_v, acc_sh,
             s0, s1, s2, s3, s4, si):
    c = lax.axis_index("c")
    s = lax.axis_index("s")
    bufs = (b0, b1, b2, b3, b4)
    sems = (s0, s1, s2, s3, s4)

    nseg = jnp.where(c == 0, SEG_PER_CORE, NSEG - SEG_PER_CORE)
    k0 = c * SEG_PER_CORE

    zero16 = jnp.zeros((NLANE,), jnp.float32)

    def _zero(kk, carry):
        for u in range(NVEC):
            acc_v[kk, pl.ds(u * NLANE, NLANE)] = zero16
        return carry

    lax.fori_loop(0, NSUB, _zero, 0)

    @pl.when(s == 0)
    def _():
        pltpu.sync_copy(acc_v, acc_sh)

    plsc.subcore_barrier()

    lanes = lax.iota(jnp.int32, NLANE)
    lo = lanes < 8
    ia = jnp.minimum(lanes * 2, 15)
    ja = jnp.minimum(lanes * 2 + 1, 15)
    ib = jnp.clip(lanes * 2 - 16, 0, 15)
    jb = jnp.clip(lanes * 2 - 15, 0, 15)
    ib7 = jnp.clip(lanes * 2 - 10, 0, 15)
    jb7 = jnp.clip(lanes * 2 - 9, 0, 15)

    def compute_idx(pb):
        for ch in range(NCHUNK):
            for v in range(IVEC):
                a = prs_v[pb, ch, pl.ds(2 * NLANE * v, NLANE)]
                boff = 2 * NLANE * v + NLANE if v < IVEC - 1 else PROW - NLANE
                b = prs_v[pb, ch, pl.ds(boff, NLANE)]
                ibv, jbv = (ib, jb) if v < IVEC - 1 else (ib7, jb7)
                iv = jnp.where(lo, _take16(a, ia), _take16(b, ibv))
                jv = jnp.where(lo, _take16(a, ja), _take16(b, jbv))
                idx_v[pb, ch, pl.ds(v * NLANE, NLANE)] = iv * COLS + jv

    def stage_pairs(k, pb, sem):
        for ch in range(NCHUNK):
            pltpu.async_copy(pairs_hbm.at[k, s, ch], prs_v.at[pb, ch], sem)

    def drain_pairs(sem):
        for ch in range(NCHUNK):
            pltpu.make_async_copy(pairs_hbm.at[0, 0, 0],
                                  prs_v.at[0, ch], sem).wait()

    stage_pairs(k0, 0, si)
    drain_pairs(si)
    compute_idx(0)
    stage_pairs(k0 + 1, 1, si)

    for b in range(NBUF):
        pltpu.async_copy(table_hbm.at[idx_v.at[0, b, pl.ds(0, CHUNK)]],
                         bufs[b], sems[b])

    def seg_body(kk, carry):
        pb = kk & 1

        @pl.when(kk + 1 < nseg)
        def _():
            drain_pairs(si)
            compute_idx(1 - pb)

            @pl.when(kk + 2 < nseg)
            def _():
                stage_pairs(k0 + kk + 2, pb, si)

        for ch in range(NCHUNK):
            buf, sem = bufs[ch % NBUF], sems[ch % NBUF]
            pltpu.make_async_copy(
                table_hbm.at[idx_v.at[0, 0, pl.ds(0, CHUNK)]],
                buf, sem).wait()

            def row_body(i, a):
                out = a
                for r in range(ROW_UNROLL):
                    row = i * ROW_UNROLL + r
                    out = tuple(
                        out[u] + buf[row, pl.ds(u * NLANE, NLANE)]
                        for u in range(NVEC)
                    )
                return out

            acc = lax.fori_loop(0, CHUNK // ROW_UNROLL, row_body,
                                tuple(zero16 for _ in range(NVEC)))
            for u in range(NVEC):
                plsc.addupdate(acc_v.at[kk, pl.ds(u * NLANE, NLANE)], acc[u])

            if ch < NCHUNK - NBUF:
                pltpu.async_copy(
                    table_hbm.at[idx_v.at[pb, ch + NBUF, pl.ds(0, CHUNK)]],
                    buf, sem)
            else:
                @pl.when(kk + 1 < nseg)
                def _():
                    pltpu.async_copy(
                        table_hbm.at[
                            idx_v.at[1 - pb, ch + NBUF - NCHUNK,
                                     pl.ds(0, CHUNK)]],
                        buf, sem)
        return carry

    lax.fori_loop(0, nseg, seg_body, 0)

    row_ids = lax.iota(jnp.int32, NLANE)
    pltpu.sync_copy(acc_v, acc_sh.at[row_ids], add=True)
    plsc.subcore_barrier()

    @pl.when(s == 0)
    def _():
        pltpu.sync_copy(acc_sh, acc_v)
        inv = jnp.full((NLANE,), 1.0 / NPAIR, jnp.float32)

        def scale_body(kk, carry):
            for u in range(NVEC):
                sl = pl.ds(u * NLANE, NLANE)
                acc_v[kk, sl] = acc_v[kk, sl] * inv
            return carry

        lax.fori_loop(0, NSUB, scale_body, 0)
        pltpu.sync_copy(acc_v, out_hbm.at[c])


@jax.jit
def _node_pool_sc(table, pairs4):
    mesh = plsc.VectorSubcoreMesh(core_axis_name="c", subcore_axis_name="s")
    k = functools.partial(
        pl.kernel,
        out_type=jax.ShapeDtypeStruct((NCORE, NSUB, UNITS), jnp.float32),
        mesh=mesh,
        scratch_types=[
            pltpu.VMEM((2, NCHUNK, PROW), jnp.int32),
            pltpu.VMEM((2, NCHUNK, IROW), jnp.int32),
            pltpu.VMEM((CHUNK, UNITS), jnp.float32),
            pltpu.VMEM((CHUNK, UNITS), jnp.float32),
            pltpu.VMEM((CHUNK, UNITS), jnp.float32),
            pltpu.VMEM((CHUNK, UNITS), jnp.float32),
            pltpu.VMEM((CHUNK, UNITS), jnp.float32),
            pltpu.VMEM((NSUB, UNITS), jnp.float32),
            pltpu.VMEM_SHARED((NSUB, UNITS), jnp.float32),
            pltpu.SemaphoreType.DMA,
            pltpu.SemaphoreType.DMA,
            pltpu.SemaphoreType.DMA,
            pltpu.SemaphoreType.DMA,
            pltpu.SemaphoreType.DMA,
            pltpu.SemaphoreType.DMA,
        ],
    )(_sc_body)
    return k(table, pairs4)


def kernel(inputs, pairs):
    table = inputs.reshape(ROWS * COLS, UNITS)
    pairs4 = pairs.reshape(NSEG, NSUB, NCHUNK, PROW)
    out = _node_pool_sc(table, pairs4)
    return jnp.concatenate(
        [out[0, :SEG_PER_CORE], out[1, :NSEG - SEG_PER_CORE]], axis=0)


# speedup vs baseline: 1.6633x; 1.6633x over previous
import functools

import jax
import jax.numpy as jnp
from jax import lax
from jax.experimental import pallas as pl
from jax.experimental.pallas import tpu as pltpu
from jax.experimental.pallas import tpu_sc as plsc

NSEG = 27
NPAIR = 20000
UNITS = 128
ROWS = 512
COLS = 1024

NCORE = 2
NSUB = 16
SEG_PER_CORE = 14
NCHUNK = 10
CHUNK = 125
NLANE = 16
NVEC = UNITS // NLANE
NBUF = 5
ROW_UNROLL = 5


def _sc_body(table_hbm, idx_hbm, out_hbm,
             idx_v, b0, b1, b2, b3, b4, acc_v, acc_sh,
             s0, s1, s2, s3, s4, si):
    c = lax.axis_index("c")
    s = lax.axis_index("s")
    bufs = (b0, b1, b2, b3, b4)
    sems = (s0, s1, s2, s3, s4)

    nseg = jnp.where(c == 0, SEG_PER_CORE, NSEG - SEG_PER_CORE)
    nq = nseg * NCHUNK

    zero16 = jnp.zeros((NLANE,), jnp.float32)

    def _zero(kk, carry):
        for u in range(NVEC):
            acc_v[kk, pl.ds(u * NLANE, NLANE)] = zero16
        return carry

    lax.fori_loop(0, NSUB, _zero, 0)

    @pl.when(s == 0)
    def _():
        pltpu.sync_copy(acc_v, acc_sh)

    plsc.subcore_barrier()

    def idx_start(kk, carry):
        pltpu.async_copy(idx_hbm.at[c * SEG_PER_CORE + kk, s],
                         idx_v.at[kk], si)
        return carry

    lax.fori_loop(0, nseg, idx_start, 0)

    def idx_wait(kk, carry):
        pltpu.make_async_copy(idx_hbm.at[0, 0], idx_v.at[kk], si).wait()
        return carry

    lax.fori_loop(0, nseg, idx_wait, 0)

    for b in range(NBUF):
        pltpu.async_copy(table_hbm.at[idx_v.at[0, b]], bufs[b], sems[b])

    def q_body(g, carry):
        for b in range(NBUF):
            q = g * NBUF + b
            buf, sem = bufs[b], sems[b]
            pltpu.make_async_copy(table_hbm.at[idx_v.at[0, 0]],
                                  buf, sem).wait()

            kk = (q * 6554) >> 16

            def row_body(i, a):
                out = a
                for r in range(ROW_UNROLL):
                    row = i * ROW_UNROLL + r
                    out = tuple(
                        out[u] + buf[row, pl.ds(u * NLANE, NLANE)]
                        for u in range(NVEC)
                    )
                return out

            acc = lax.fori_loop(0, CHUNK // ROW_UNROLL, row_body,
                                tuple(zero16 for _ in range(NVEC)))
            for u in range(NVEC):
                plsc.addupdate(acc_v.at[kk, pl.ds(u * NLANE, NLANE)], acc[u])

            qn = q + NBUF

            @pl.when(qn < nq)
            def _():
                kk2 = (qn * 6554) >> 16
                ch2 = qn - kk2 * NCHUNK
                pltpu.async_copy(table_hbm.at[idx_v.at[kk2, ch2]], buf, sem)
        return carry

    lax.fori_loop(0, nq // NBUF, q_body, 0)

    row_ids = lax.iota(jnp.int32, NLANE)
    pltpu.sync_copy(acc_v, acc_sh.at[row_ids], add=True)
    plsc.subcore_barrier()

    @pl.when(s == 0)
    def _():
        pltpu.sync_copy(acc_sh, acc_v)
        inv = jnp.full((NLANE,), 1.0 / NPAIR, jnp.float32)

        def scale_body(kk, carry):
            for u in range(NVEC):
                sl = pl.ds(u * NLANE, NLANE)
                acc_v[kk, sl] = acc_v[kk, sl] * inv
            return carry

        lax.fori_loop(0, NSUB, scale_body, 0)
        pltpu.sync_copy(acc_v, out_hbm.at[c])


@jax.jit
def _node_pool_sc(table, idx4):
    mesh = plsc.VectorSubcoreMesh(core_axis_name="c", subcore_axis_name="s")
    k = functools.partial(
        pl.kernel,
        out_type=jax.ShapeDtypeStruct((NCORE, NSUB, UNITS), jnp.float32),
        mesh=mesh,
        scratch_types=[
            pltpu.VMEM((SEG_PER_CORE, NCHUNK, CHUNK), jnp.int32),
            pltpu.VMEM((CHUNK, UNITS), jnp.float32),
            pltpu.VMEM((CHUNK, UNITS), jnp.float32),
            pltpu.VMEM((CHUNK, UNITS), jnp.float32),
            pltpu.VMEM((CHUNK, UNITS), jnp.float32),
            pltpu.VMEM((CHUNK, UNITS), jnp.float32),
            pltpu.VMEM((NSUB, UNITS), jnp.float32),
            pltpu.VMEM_SHARED((NSUB, UNITS), jnp.float32),
            pltpu.SemaphoreType.DMA,
            pltpu.SemaphoreType.DMA,
            pltpu.SemaphoreType.DMA,
            pltpu.SemaphoreType.DMA,
            pltpu.SemaphoreType.DMA,
            pltpu.SemaphoreType.DMA,
        ],
    )(_sc_body)
    return k(table, idx4)


def kernel(inputs, pairs):
    table = inputs.reshape(ROWS * COLS, UNITS)
    p4 = pairs.reshape(NSEG, NSUB, NCHUNK, CHUNK, 2)
    idx = p4[..., 0] * COLS + p4[..., 1]
    out = _node_pool_sc(table, idx)
    return jnp.concatenate(
        [out[0, :SEG_PER_CORE], out[1, :NSEG - SEG_PER_CORE]], axis=0)
